# baseline (speedup 1.0000x reference)
import functools
import math

import jax
import jax.numpy as jnp
from jax import lax
from jax.experimental import pallas as pl
from jax.experimental.pallas import tpu as pltpu
from jax.experimental.pallas import tpu_sc as plsc

VOCAB = 100000
SEQ = 2048
D_MODEL = 768
PAD_ID = 0
BATCH = 4

B_TOTAL = BATCH * SEQ
NC, NS = 2, 16
NW = NC * NS
B_PER_W = B_TOTAL // NW
CHUNK = 128
N_CHUNKS = B_PER_W // CHUNK

_SCALE = 1.0 / math.sqrt(D_MODEL)


def _sc_gather(table, idx_flat):
    mesh = plsc.VectorSubcoreMesh(core_axis_name="c", subcore_axis_name="s")

    @functools.partial(
        pl.kernel,
        mesh=mesh,
        out_type=jax.ShapeDtypeStruct((B_TOTAL, D_MODEL), jnp.float32),
        scratch_types=[
            pltpu.VMEM((B_PER_W,), jnp.int32),
            pltpu.VMEM((CHUNK, D_MODEL), jnp.float32),
            pltpu.SemaphoreType.DMA,
        ],
    )
    def k(table_hbm, idx_hbm, out_hbm, idx_v, rows_v, sem):
        wid = lax.axis_index("s") * NC + lax.axis_index("c")
        base = wid * B_PER_W
        pltpu.sync_copy(idx_hbm.at[pl.ds(base, B_PER_W)], idx_v)
        for c in range(N_CHUNKS):
            pltpu.async_copy(
                table_hbm.at[idx_v.at[pl.ds(c * CHUNK, CHUNK)]], rows_v, sem
            ).wait()
            pltpu.sync_copy(rows_v, out_hbm.at[pl.ds(base + c * CHUNK, CHUNK)])

    return k(table, idx_flat)


def _tc_fuse_body(x_ref, g_ref, p_ref, o_ref):
    scale_row = jnp.where(x_ref[0, 0, 0] != PAD_ID, _SCALE, 0.0)
    o_ref[0] = g_ref[0] * scale_row.reshape(_BLK_S, 1) + p_ref[...]


_BLK_S = 128


def _tc_fuse(gathered, x_flat, pos_table):
    n_s = SEQ // _BLK_S
    g4 = gathered.reshape(BATCH, SEQ, D_MODEL)
    x4 = x_flat.reshape(BATCH, n_s, 1, _BLK_S)
    out = pl.pallas_call(
        _tc_fuse_body,
        grid=(n_s, BATCH),
        in_specs=[
            pl.BlockSpec((1, 1, 1, _BLK_S), lambda s, b: (b, s, 0, 0)),
            pl.BlockSpec((1, _BLK_S, D_MODEL), lambda s, b: (b, s, 0)),
            pl.BlockSpec((_BLK_S, D_MODEL), lambda s, b: (s, 0)),
        ],
        out_specs=pl.BlockSpec((1, _BLK_S, D_MODEL), lambda s, b: (b, s, 0)),
        out_shape=jax.ShapeDtypeStruct((BATCH, SEQ, D_MODEL), jnp.float32),
        compiler_params=pltpu.CompilerParams(
            dimension_semantics=("parallel", "parallel")
        ),
    )(x4, g4, pos_table)
    return out


def kernel(x, tok_table, pos_table):
    x_flat = x.astype(jnp.int32).reshape(B_TOTAL)
    gathered = _sc_gather(tok_table, x_flat)
    return _tc_fuse(gathered, x_flat, pos_table)


# speedup vs baseline: 1.4585x; 1.4585x over previous
import functools
import math

import jax
import jax.numpy as jnp
from jax import lax
from jax.experimental import pallas as pl
from jax.experimental.pallas import tpu as pltpu
from jax.experimental.pallas import tpu_sc as plsc

VOCAB = 100000
SEQ = 2048
D_MODEL = 768
PAD_ID = 0
BATCH = 4

B_TOTAL = BATCH * SEQ
NC, NS = 2, 16
NW = NC * NS
B_PER_W = B_TOTAL // NW
CHUNK = 128
N_CHUNKS = B_PER_W // CHUNK

_SCALE = 1.0 / math.sqrt(D_MODEL)


def _sc_gather(table, idx_flat):
    mesh = plsc.VectorSubcoreMesh(core_axis_name="c", subcore_axis_name="s")

    @functools.partial(
        pl.kernel,
        mesh=mesh,
        out_type=jax.ShapeDtypeStruct((B_TOTAL, D_MODEL), jnp.float32),
        scratch_types=[
            pltpu.VMEM((B_PER_W,), jnp.int32),
            pltpu.VMEM((CHUNK, D_MODEL), jnp.float32),
            pltpu.SemaphoreType.DMA,
        ],
    )
    def k(table_hbm, idx_hbm, out_hbm, idx_v, rows_v, sem):
        wid = lax.axis_index("s") * NC + lax.axis_index("c")
        base = wid * B_PER_W
        pltpu.sync_copy(idx_hbm.at[pl.ds(base, B_PER_W)], idx_v)
        for c in range(N_CHUNKS):
            pltpu.async_copy(
                table_hbm.at[idx_v.at[pl.ds(c * CHUNK, CHUNK)]], rows_v, sem
            ).wait()
            pltpu.sync_copy(rows_v, out_hbm.at[pl.ds(base + c * CHUNK, CHUNK)])

    return k(table, idx_flat)


def _tc_fuse_body(x_ref, g_ref, p_ref, o_ref):
    scale_row = jnp.where(x_ref[0, 0, 0] != PAD_ID, _SCALE, 0.0)
    o_ref[0] = g_ref[0] * scale_row.reshape(_BLK_S, 1) + p_ref[...]


_BLK_S = 1024


def _tc_fuse(gathered, x_flat, pos_table):
    n_s = SEQ // _BLK_S
    g4 = gathered.reshape(BATCH, SEQ, D_MODEL)
    x4 = x_flat.reshape(BATCH, n_s, 1, _BLK_S)
    out = pl.pallas_call(
        _tc_fuse_body,
        grid=(n_s, BATCH),
        in_specs=[
            pl.BlockSpec((1, 1, 1, _BLK_S), lambda s, b: (b, s, 0, 0)),
            pl.BlockSpec((1, _BLK_S, D_MODEL), lambda s, b: (b, s, 0)),
            pl.BlockSpec((_BLK_S, D_MODEL), lambda s, b: (s, 0)),
        ],
        out_specs=pl.BlockSpec((1, _BLK_S, D_MODEL), lambda s, b: (b, s, 0)),
        out_shape=jax.ShapeDtypeStruct((BATCH, SEQ, D_MODEL), jnp.float32),
        compiler_params=pltpu.CompilerParams(
            dimension_semantics=("parallel", "parallel")
        ),
    )(x4, g4, pos_table)
    return out


def kernel(x, tok_table, pos_table):
    x_flat = x.astype(jnp.int32).reshape(B_TOTAL)
    gathered = _sc_gather(tok_table, x_flat)
    return _tc_fuse(gathered, x_flat, pos_table)


# speedup vs baseline: 1.4927x; 1.0235x over previous
import functools
import math

import jax
import jax.numpy as jnp
from jax import lax
from jax.experimental import pallas as pl
from jax.experimental.pallas import tpu as pltpu
from jax.experimental.pallas import tpu_sc as plsc

VOCAB = 100000
SEQ = 2048
D_MODEL = 768
PAD_ID = 0
BATCH = 4

B_TOTAL = BATCH * SEQ
NC, NS = 2, 16
NW = NC * NS
B_PER_W = B_TOTAL // NW
CHUNK = 128
N_CHUNKS = B_PER_W // CHUNK

_SCALE = 1.0 / math.sqrt(D_MODEL)


def _sc_gather(table, idx_flat):
    mesh = plsc.VectorSubcoreMesh(core_axis_name="c", subcore_axis_name="s")

    @functools.partial(
        pl.kernel,
        mesh=mesh,
        out_type=jax.ShapeDtypeStruct((B_TOTAL, D_MODEL), jnp.float32),
        scratch_types=[
            pltpu.VMEM((B_PER_W,), jnp.int32),
            pltpu.VMEM((CHUNK, D_MODEL), jnp.float32),
            pltpu.SemaphoreType.DMA,
        ],
    )
    def k(table_hbm, idx_hbm, out_hbm, idx_v, rows_v, sem):
        wid = lax.axis_index("s") * NC + lax.axis_index("c")
        base = wid * B_PER_W
        pltpu.sync_copy(idx_hbm.at[pl.ds(base, B_PER_W)], idx_v)
        for c in range(N_CHUNKS):
            pltpu.async_copy(
                table_hbm.at[idx_v.at[pl.ds(c * CHUNK, CHUNK)]], rows_v, sem
            ).wait()
            pltpu.sync_copy(rows_v, out_hbm.at[pl.ds(base + c * CHUNK, CHUNK)])

    return k(table, idx_flat)


def _tc_fuse_body(x_ref, g_ref, p_ref, o_ref):
    scale_row = jnp.where(x_ref[0, 0, 0] != PAD_ID, _SCALE, 0.0)
    o_ref[0] = g_ref[0] * scale_row.reshape(_BLK_S, 1) + p_ref[...]


_BLK_S = 2048


def _tc_fuse(gathered, x_flat, pos_table):
    n_s = SEQ // _BLK_S
    g4 = gathered.reshape(BATCH, SEQ, D_MODEL)
    x4 = x_flat.reshape(BATCH, n_s, 1, _BLK_S)
    out = pl.pallas_call(
        _tc_fuse_body,
        grid=(n_s, BATCH),
        in_specs=[
            pl.BlockSpec((1, 1, 1, _BLK_S), lambda s, b: (b, s, 0, 0)),
            pl.BlockSpec((1, _BLK_S, D_MODEL), lambda s, b: (b, s, 0)),
            pl.BlockSpec((_BLK_S, D_MODEL), lambda s, b: (s, 0)),
        ],
        out_specs=pl.BlockSpec((1, _BLK_S, D_MODEL), lambda s, b: (b, s, 0)),
        out_shape=jax.ShapeDtypeStruct((BATCH, SEQ, D_MODEL), jnp.float32),
        compiler_params=pltpu.CompilerParams(
            dimension_semantics=("parallel", "parallel")
        ),
    )(x4, g4, pos_table)
    return out


def kernel(x, tok_table, pos_table):
    x_flat = x.astype(jnp.int32).reshape(B_TOTAL)
    gathered = _sc_gather(tok_table, x_flat)
    return _tc_fuse(gathered, x_flat, pos_table)
